# baseline (speedup 1.0000x reference)
import jax
import jax.numpy as jnp
from jax.experimental import pallas as pl

_N = 10000
_E = 320000
_D = 128
_BE = 4000
_BN = 2000
_GE = _E // _BE
_GN = _N // _BN


def _rels_body(rels_ref, kb0_ref, kc0_ref, kb1_ref, kc1_ref,
               rhat_ref, q0_ref, c0_ref, q1_ref, c1_ref):
    rels = rels_ref[...]
    norm = jnp.maximum(jnp.sum(jnp.abs(rels), axis=1, keepdims=True), 1e-12)
    rhat = rels / norm
    rhat_ref[...] = rhat
    q0_ref[...] = jnp.sum(rhat * kb0_ref[...], axis=1, keepdims=True)
    c0_ref[...] = jnp.sum(rhat * kc0_ref[...], axis=1, keepdims=True)
    q1_ref[...] = jnp.sum(rhat * kb1_ref[...], axis=1, keepdims=True)
    c1_ref[...] = jnp.sum(rhat * kc1_ref[...], axis=1, keepdims=True)


def _edge_a_body(fsrc_ref, rhat_ref, udst_ref, vsrc_ref, q_ref, c_ref,
                 p_ref, e_ref):
    fsrc = fsrc_ref[...]
    rhat = rhat_ref[...]
    p = jnp.sum(fsrc * rhat, axis=1, keepdims=True)
    att1 = udst_ref[...] + vsrc_ref[...] - 2.0 * p * q_ref[...] + c_ref[...]
    p_ref[...] = p
    e_ref[...] = jnp.exp(att1)


def _edge_c_body(fsrc_ref, rhat_ref, e_ref, dend_ref, p_ref, out_ref):
    w = e_ref[...] / dend_ref[...]
    out_ref[...] = w * fsrc_ref[...] - (2.0 * w * p_ref[...]) * rhat_ref[...]


def _node0_body(acc_ref, cnt_ref, ka_ref, kb_ref, f_ref, u_ref, v_ref):
    f = jnp.maximum(acc_ref[...] / jnp.maximum(cnt_ref[...], 1.0), 0.0)
    f_ref[...] = f
    u_ref[...] = jnp.sum(f * ka_ref[...], axis=1, keepdims=True)
    v_ref[...] = jnp.sum(f * kb_ref[...], axis=1, keepdims=True)


def _node_body(acc_ref, ka_ref, kb_ref, f_ref, u_ref, v_ref):
    f = jnp.maximum(acc_ref[...], 0.0)
    f_ref[...] = f
    u_ref[...] = jnp.sum(f * ka_ref[...], axis=1, keepdims=True)
    v_ref[...] = jnp.sum(f * kb_ref[...], axis=1, keepdims=True)


_vecE = pl.BlockSpec((_BE, _D), lambda i: (i, 0))
_scaE = pl.BlockSpec((_BE, 1), lambda i: (i, 0))
_vecN = pl.BlockSpec((_BN, _D), lambda i: (i, 0))
_scaN = pl.BlockSpec((_BN, 1), lambda i: (i, 0))
_wsp = pl.BlockSpec((1, _D), lambda i: (0, 0))

_fE = jax.ShapeDtypeStruct((_E, _D), jnp.float32)
_sE = jax.ShapeDtypeStruct((_E, 1), jnp.float32)
_fN = jax.ShapeDtypeStruct((_N, _D), jnp.float32)
_sN = jax.ShapeDtypeStruct((_N, 1), jnp.float32)


def _rels_prep(rels, kb0, kc0, kb1, kc1):
    return pl.pallas_call(
        _rels_body,
        grid=(_GE,),
        in_specs=[_vecE, _wsp, _wsp, _wsp, _wsp],
        out_specs=[_vecE, _scaE, _scaE, _scaE, _scaE],
        out_shape=[_fE, _sE, _sE, _sE, _sE],
    )(rels, kb0, kc0, kb1, kc1)


def _edge_a(fsrc, rhat, udst, vsrc, q, c):
    return pl.pallas_call(
        _edge_a_body,
        grid=(_GE,),
        in_specs=[_vecE, _vecE, _scaE, _scaE, _scaE, _scaE],
        out_specs=[_scaE, _scaE],
        out_shape=[_sE, _sE],
    )(fsrc, rhat, udst, vsrc, q, c)


def _edge_c(fsrc, rhat, e, dend, p):
    return pl.pallas_call(
        _edge_c_body,
        grid=(_GE,),
        in_specs=[_vecE, _vecE, _scaE, _scaE, _scaE],
        out_specs=_vecE,
        out_shape=_fE,
    )(fsrc, rhat, e, dend, p)


def _node0(acc, cnt, ka, kb):
    return pl.pallas_call(
        _node0_body,
        grid=(_GN,),
        in_specs=[_vecN, _scaN, _wsp, _wsp],
        out_specs=[_vecN, _scaN, _scaN],
        out_shape=[_fN, _sN, _sN],
    )(acc, cnt, ka, kb)


def _node(acc, ka, kb):
    return pl.pallas_call(
        _node_body,
        grid=(_GN,),
        in_specs=[_vecN, _wsp, _wsp],
        out_specs=[_vecN, _scaN, _scaN],
        out_shape=[_fN, _sN, _sN],
    )(acc, ka, kb)


def kernel(adj_input, r_index, r_val, t_index, ent_matrix, rel_matrix,
           time_matrix, ent_emb_r, ent_emb_t, rel_emb, time_emb,
           ak_e0, ak_e1, ak_t0, ak_t1):
    dst = adj_input[:, 0]
    src = adj_input[:, 1]

    def split_k(k):
        return (k[:_D, 0][None, :], k[_D:2 * _D, 0][None, :],
                k[2 * _D:, 0][None, :])

    ka_e0, kb_e0, kc_e0 = split_k(ak_e0)
    ka_e1, kb_e1, kc_e1 = split_k(ak_e1)
    ka_t0, kb_t0, kc_t0 = split_k(ak_t0)
    ka_t1, kb_t1, kc_t1 = split_k(ak_t1)

    def rels_of(sp_idx, emb, kb0, kc0, kb1, kc1):
        contrib = r_val[:, None] * jnp.take(emb, sp_idx[:, 1], axis=0)
        rels = jax.ops.segment_sum(contrib, sp_idx[:, 0], num_segments=_E)
        return _rels_prep(rels, kb0, kc0, kb1, kc1)

    rhat_e, qe0, ce0, qe1, ce1 = rels_of(r_index, rel_emb,
                                         kb_e0, kc_e0, kb_e1, kc_e1)
    rhat_t, qt0, ct0, qt1, ct1 = rels_of(t_index, time_emb,
                                         kb_t0, kc_t0, kb_t1, kc_t1)

    def init_feature(idx, X, ka, kb):
        rows = idx[:, 0]
        cols = idx[:, 1]
        cnt = jax.ops.segment_sum(jnp.ones((_E,), jnp.float32), rows,
                                  num_segments=_N)
        acc = jax.ops.segment_sum(jnp.take(X, cols, axis=0), rows,
                                  num_segments=_N)
        return _node0(acc, cnt[:, None], ka, kb)

    def attention(F, u, v, rhat, qs, cs, kas, kbs):
        outs = [F]
        for l in range(2):
            Fsrc = jnp.take(F, src, axis=0)
            udst = jnp.take(u, dst, axis=0)
            vsrc = jnp.take(v, src, axis=0)
            p, e = _edge_a(Fsrc, rhat, udst, vsrc, qs[l], cs[l])
            den = jax.ops.segment_sum(e[:, 0], dst, num_segments=_N)
            dend = jnp.take(den, dst, axis=0)[:, None]
            oute = _edge_c(Fsrc, rhat, e, dend, p)
            acc = jax.ops.segment_sum(oute, dst, num_segments=_N)
            nl = min(l + 1, 1)
            F, u, v = _node(acc, kas[nl], kbs[nl])
            outs.append(F)
        return jnp.concatenate(outs, axis=1)

    Fer, uer, ver = init_feature(ent_matrix, ent_emb_r, ka_e0, kb_e0)
    Frr, urr, vrr = init_feature(rel_matrix, rel_emb, ka_e0, kb_e0)
    Fet, uet, vet = init_feature(ent_matrix, ent_emb_t, ka_t0, kb_t0)
    Ftt, utt, vtt = init_feature(time_matrix, time_emb, ka_t0, kb_t0)

    ent_r = attention(Fer, uer, ver, rhat_e, (qe0, qe1), (ce0, ce1),
                      (ka_e0, ka_e1), (kb_e0, kb_e1))
    r_f = attention(Frr, urr, vrr, rhat_e, (qe0, qe1), (ce0, ce1),
                    (ka_e0, ka_e1), (kb_e0, kb_e1))
    ent_t = attention(Fet, uet, vet, rhat_t, (qt0, qt1), (ct0, ct1),
                      (ka_t0, ka_t1), (kb_t0, kb_t1))
    t_f = attention(Ftt, utt, vtt, rhat_t, (qt0, qt1), (ct0, ct1),
                    (ka_t0, ka_t1), (kb_t0, kb_t1))

    output_e_r = jnp.concatenate([ent_r, r_f], axis=-1)
    output_e_t = jnp.concatenate([ent_t, t_f], axis=-1)
    return (output_e_r, output_e_t)


# speedup vs baseline: 1.0727x; 1.0727x over previous
import jax
import jax.numpy as jnp
from jax.experimental import pallas as pl

_N = 10000
_E = 320000
_D = 128
_L = 64
_EG = _E // _L
_BG = 40
_GE = _EG // _BG
_BN = 2000
_GN = _N // _BN


def _rels_body(rels_ref, kb0_ref, kc0_ref, kb1_ref, kc1_ref,
               rhat_ref, q0_ref, c0_ref, q1_ref, c1_ref):
    rels = rels_ref[...]
    norm = jnp.maximum(jnp.sum(jnp.abs(rels), axis=2), 1e-12)
    rhat = rels / norm[..., None]
    rhat_ref[...] = rhat
    kb0 = kb0_ref[...][None]
    kc0 = kc0_ref[...][None]
    kb1 = kb1_ref[...][None]
    kc1 = kc1_ref[...][None]
    q0_ref[...] = jnp.sum(rhat * kb0, axis=2)
    c0_ref[...] = jnp.sum(rhat * kc0, axis=2)
    q1_ref[...] = jnp.sum(rhat * kb1, axis=2)
    c1_ref[...] = jnp.sum(rhat * kc1, axis=2)


def _edge_a_body(fsrc_ref, rhat_ref, udst_ref, vsrc_ref, q_ref, c_ref,
                 p_ref, e_ref):
    p = jnp.sum(fsrc_ref[...] * rhat_ref[...], axis=2)
    att1 = udst_ref[...] + vsrc_ref[...] - 2.0 * p * q_ref[...] + c_ref[...]
    p_ref[...] = p
    e_ref[...] = jnp.exp(att1)


def _edge_c_body(fsrc_ref, rhat_ref, e_ref, dend_ref, p_ref, out_ref):
    w = e_ref[...] / dend_ref[...]
    out_ref[...] = (w[..., None] * fsrc_ref[...]
                    - (2.0 * w * p_ref[...])[..., None] * rhat_ref[...])


def _node0_body(acc_ref, cnt_ref, ka_ref, kb_ref, f_ref, u_ref, v_ref):
    f = jnp.maximum(acc_ref[...] / jnp.maximum(cnt_ref[...], 1.0), 0.0)
    f_ref[...] = f
    u_ref[...] = jnp.sum(f * ka_ref[...], axis=1, keepdims=True)
    v_ref[...] = jnp.sum(f * kb_ref[...], axis=1, keepdims=True)


def _node_body(acc_ref, ka_ref, kb_ref, f_ref, u_ref, v_ref):
    f = jnp.maximum(acc_ref[...], 0.0)
    f_ref[...] = f
    u_ref[...] = jnp.sum(f * ka_ref[...], axis=1, keepdims=True)
    v_ref[...] = jnp.sum(f * kb_ref[...], axis=1, keepdims=True)


_vecE = pl.BlockSpec((_BG, _L, _D), lambda i: (i, 0, 0))
_scaE = pl.BlockSpec((_BG, _L), lambda i: (i, 0))
_vecN = pl.BlockSpec((_BN, _D), lambda i: (i, 0))
_scaN = pl.BlockSpec((_BN, 1), lambda i: (i, 0))
_wsp = pl.BlockSpec((1, _D), lambda i: (0, 0))

_fE = jax.ShapeDtypeStruct((_EG, _L, _D), jnp.float32)
_sE = jax.ShapeDtypeStruct((_EG, _L), jnp.float32)
_fN = jax.ShapeDtypeStruct((_N, _D), jnp.float32)
_sN = jax.ShapeDtypeStruct((_N, 1), jnp.float32)


def _rels_prep(rels, kb0, kc0, kb1, kc1):
    return pl.pallas_call(
        _rels_body,
        grid=(_GE,),
        in_specs=[_vecE, _wsp, _wsp, _wsp, _wsp],
        out_specs=[_vecE, _scaE, _scaE, _scaE, _scaE],
        out_shape=[_fE, _sE, _sE, _sE, _sE],
    )(rels, kb0, kc0, kb1, kc1)


def _edge_a(fsrc, rhat, udst, vsrc, q, c):
    return pl.pallas_call(
        _edge_a_body,
        grid=(_GE,),
        in_specs=[_vecE, _vecE, _scaE, _scaE, _scaE, _scaE],
        out_specs=[_scaE, _scaE],
        out_shape=[_sE, _sE],
    )(fsrc, rhat, udst, vsrc, q, c)


def _edge_c(fsrc, rhat, e, dend, p):
    return pl.pallas_call(
        _edge_c_body,
        grid=(_GE,),
        in_specs=[_vecE, _vecE, _scaE, _scaE, _scaE],
        out_specs=_vecE,
        out_shape=_fE,
    )(fsrc, rhat, e, dend, p)


def _node0(acc, cnt, ka, kb):
    return pl.pallas_call(
        _node0_body,
        grid=(_GN,),
        in_specs=[_vecN, _scaN, _wsp, _wsp],
        out_specs=[_vecN, _scaN, _scaN],
        out_shape=[_fN, _sN, _sN],
    )(acc, cnt, ka, kb)


def _node(acc, ka, kb):
    return pl.pallas_call(
        _node_body,
        grid=(_GN,),
        in_specs=[_vecN, _wsp, _wsp],
        out_specs=[_vecN, _scaN, _scaN],
        out_shape=[_fN, _sN, _sN],
    )(acc, ka, kb)


def kernel(adj_input, r_index, r_val, t_index, ent_matrix, rel_matrix,
           time_matrix, ent_emb_r, ent_emb_t, rel_emb, time_emb,
           ak_e0, ak_e1, ak_t0, ak_t1):
    dst = adj_input[:, 0]
    src = adj_input[:, 1]

    def split_k(k):
        return (k[:_D, 0][None, :], k[_D:2 * _D, 0][None, :],
                k[2 * _D:, 0][None, :])

    ka_e0, kb_e0, kc_e0 = split_k(ak_e0)
    ka_e1, kb_e1, kc_e1 = split_k(ak_e1)
    ka_t0, kb_t0, kc_t0 = split_k(ak_t0)
    ka_t1, kb_t1, kc_t1 = split_k(ak_t1)

    def rels_of(sp_idx, emb, kb0, kc0, kb1, kc1):
        contrib = r_val[:, None] * jnp.take(emb, sp_idx[:, 1], axis=0)
        rels = jax.ops.segment_sum(contrib, sp_idx[:, 0], num_segments=_E)
        return _rels_prep(rels.reshape(_EG, _L, _D), kb0, kc0, kb1, kc1)

    rhat_e, qe0, ce0, qe1, ce1 = rels_of(r_index, rel_emb,
                                         kb_e0, kc_e0, kb_e1, kc_e1)
    rhat_t, qt0, ct0, qt1, ct1 = rels_of(t_index, time_emb,
                                         kb_t0, kc_t0, kb_t1, kc_t1)

    def init_feature(idx, X, ka, kb):
        rows = idx[:, 0]
        cols = idx[:, 1]
        cnt = jax.ops.segment_sum(jnp.ones((_E,), jnp.float32), rows,
                                  num_segments=_N)
        acc = jax.ops.segment_sum(jnp.take(X, cols, axis=0), rows,
                                  num_segments=_N)
        return _node0(acc, cnt[:, None], ka, kb)

    def attention(F, u, v, rhat, qs, cs, kas, kbs):
        outs = [F]
        for l in range(2):
            Fsrc = jnp.take(F, src, axis=0).reshape(_EG, _L, _D)
            udst = jnp.take(u[:, 0], dst, axis=0).reshape(_EG, _L)
            vsrc = jnp.take(v[:, 0], src, axis=0).reshape(_EG, _L)
            p, e = _edge_a(Fsrc, rhat, udst, vsrc, qs[l], cs[l])
            den = jax.ops.segment_sum(e.reshape(_E), dst, num_segments=_N)
            dend = jnp.take(den, dst, axis=0).reshape(_EG, _L)
            oute = _edge_c(Fsrc, rhat, e, dend, p)
            acc = jax.ops.segment_sum(oute.reshape(_E, _D), dst,
                                      num_segments=_N)
            nl = min(l + 1, 1)
            F, u, v = _node(acc, kas[nl], kbs[nl])
            outs.append(F)
        return jnp.concatenate(outs, axis=1)

    Fer, uer, ver = init_feature(ent_matrix, ent_emb_r, ka_e0, kb_e0)
    Frr, urr, vrr = init_feature(rel_matrix, rel_emb, ka_e0, kb_e0)
    Fet, uet, vet = init_feature(ent_matrix, ent_emb_t, ka_t0, kb_t0)
    Ftt, utt, vtt = init_feature(time_matrix, time_emb, ka_t0, kb_t0)

    ent_r = attention(Fer, uer, ver, rhat_e, (qe0, qe1), (ce0, ce1),
                      (ka_e0, ka_e1), (kb_e0, kb_e1))
    r_f = attention(Frr, urr, vrr, rhat_e, (qe0, qe1), (ce0, ce1),
                    (ka_e0, ka_e1), (kb_e0, kb_e1))
    ent_t = attention(Fet, uet, vet, rhat_t, (qt0, qt1), (ct0, ct1),
                      (ka_t0, ka_t1), (kb_t0, kb_t1))
    t_f = attention(Ftt, utt, vtt, rhat_t, (qt0, qt1), (ct0, ct1),
                    (ka_t0, ka_t1), (kb_t0, kb_t1))

    output_e_r = jnp.concatenate([ent_r, r_f], axis=-1)
    output_e_t = jnp.concatenate([ent_t, t_f], axis=-1)
    return (output_e_r, output_e_t)


# speedup vs baseline: 1.3785x; 1.2851x over previous
import jax
import jax.numpy as jnp
from jax.experimental import pallas as pl

_N = 10000
_E = 320000
_D = 128
_L = 64
_EG = _E // _L
_BG = 40
_GE = _EG // _BG
_BN = 2000
_GN = _N // _BN


def _rels_body(rels_ref, kb0_ref, kc0_ref, kb1_ref, kc1_ref,
               rhat_ref, q0_ref, c0_ref, q1_ref, c1_ref):
    rels = rels_ref[...]
    norm = jnp.maximum(jnp.sum(jnp.abs(rels), axis=2), 1e-12)
    rhat = rels / norm[..., None]
    rhat_ref[...] = rhat
    kb0 = kb0_ref[...][None]
    kc0 = kc0_ref[...][None]
    kb1 = kb1_ref[...][None]
    kc1 = kc1_ref[...][None]
    q0_ref[...] = jnp.sum(rhat * kb0, axis=2)
    c0_ref[...] = jnp.sum(rhat * kc0, axis=2)
    q1_ref[...] = jnp.sum(rhat * kb1, axis=2)
    c1_ref[...] = jnp.sum(rhat * kc1, axis=2)


def _edge_ac_body(fsrc_ref, rhat_ref, udst_ref, vsrc_ref, q_ref, c_ref,
                  e_ref, out_ref):
    fsrc = fsrc_ref[...]
    rhat = rhat_ref[...]
    p = jnp.sum(fsrc * rhat, axis=2)
    att1 = udst_ref[...] + vsrc_ref[...] - 2.0 * p * q_ref[...] + c_ref[...]
    e = jnp.exp(att1)
    e_ref[...] = e
    out_ref[...] = e[..., None] * fsrc - (2.0 * e * p)[..., None] * rhat


def _node0_body(acc_ref, cnt_ref, ka_ref, kb_ref, f_ref, u_ref, v_ref):
    f = jnp.maximum(acc_ref[...] / jnp.maximum(cnt_ref[...], 1.0), 0.0)
    f_ref[...] = f
    u_ref[...] = jnp.sum(f * ka_ref[...], axis=1, keepdims=True)
    v_ref[...] = jnp.sum(f * kb_ref[...], axis=1, keepdims=True)


def _node_div_body(num_ref, den_ref, ka_ref, kb_ref, f_ref, u_ref, v_ref):
    f = jnp.maximum(num_ref[...] / jnp.maximum(den_ref[...], 1e-30), 0.0)
    f_ref[...] = f
    u_ref[...] = jnp.sum(f * ka_ref[...], axis=1, keepdims=True)
    v_ref[...] = jnp.sum(f * kb_ref[...], axis=1, keepdims=True)


_vecE = pl.BlockSpec((_BG, _L, _D), lambda i: (i, 0, 0))
_scaE = pl.BlockSpec((_BG, _L), lambda i: (i, 0))
_vecN = pl.BlockSpec((_BN, _D), lambda i: (i, 0))
_scaN = pl.BlockSpec((_BN, 1), lambda i: (i, 0))
_wsp = pl.BlockSpec((1, _D), lambda i: (0, 0))

_fE = jax.ShapeDtypeStruct((_EG, _L, _D), jnp.float32)
_sE = jax.ShapeDtypeStruct((_EG, _L), jnp.float32)
_fN = jax.ShapeDtypeStruct((_N, _D), jnp.float32)
_sN = jax.ShapeDtypeStruct((_N, 1), jnp.float32)


def _rels_prep(rels, kb0, kc0, kb1, kc1):
    return pl.pallas_call(
        _rels_body,
        grid=(_GE,),
        in_specs=[_vecE, _wsp, _wsp, _wsp, _wsp],
        out_specs=[_vecE, _scaE, _scaE, _scaE, _scaE],
        out_shape=[_fE, _sE, _sE, _sE, _sE],
    )(rels, kb0, kc0, kb1, kc1)


def _edge_ac(fsrc, rhat, udst, vsrc, q, c):
    return pl.pallas_call(
        _edge_ac_body,
        grid=(_GE,),
        in_specs=[_vecE, _vecE, _scaE, _scaE, _scaE, _scaE],
        out_specs=[_scaE, _vecE],
        out_shape=[_sE, _fE],
    )(fsrc, rhat, udst, vsrc, q, c)


def _node0(acc, cnt, ka, kb):
    return pl.pallas_call(
        _node0_body,
        grid=(_GN,),
        in_specs=[_vecN, _scaN, _wsp, _wsp],
        out_specs=[_vecN, _scaN, _scaN],
        out_shape=[_fN, _sN, _sN],
    )(acc, cnt, ka, kb)


def _node_div(num, den, ka, kb):
    return pl.pallas_call(
        _node_div_body,
        grid=(_GN,),
        in_specs=[_vecN, _scaN, _wsp, _wsp],
        out_specs=[_vecN, _scaN, _scaN],
        out_shape=[_fN, _sN, _sN],
    )(num, den, ka, kb)


def kernel(adj_input, r_index, r_val, t_index, ent_matrix, rel_matrix,
           time_matrix, ent_emb_r, ent_emb_t, rel_emb, time_emb,
           ak_e0, ak_e1, ak_t0, ak_t1):
    dst = adj_input[:, 0]
    src = adj_input[:, 1]

    def split_k(k):
        return (k[:_D, 0][None, :], k[_D:2 * _D, 0][None, :],
                k[2 * _D:, 0][None, :])

    ka_e0, kb_e0, kc_e0 = split_k(ak_e0)
    ka_e1, kb_e1, kc_e1 = split_k(ak_e1)
    ka_t0, kb_t0, kc_t0 = split_k(ak_t0)
    ka_t1, kb_t1, kc_t1 = split_k(ak_t1)

    def rels_of(sp_idx, emb, kb0, kc0, kb1, kc1):
        contrib = r_val[:, None] * jnp.take(emb, sp_idx[:, 1], axis=0)
        rels = jax.ops.segment_sum(contrib, sp_idx[:, 0], num_segments=_E)
        return _rels_prep(rels.reshape(_EG, _L, _D), kb0, kc0, kb1, kc1)

    rhat_e, qe0, ce0, qe1, ce1 = rels_of(r_index, rel_emb,
                                         kb_e0, kc_e0, kb_e1, kc_e1)
    rhat_t, qt0, ct0, qt1, ct1 = rels_of(t_index, time_emb,
                                         kb_t0, kc_t0, kb_t1, kc_t1)

    def init_feature(idx, X, ka, kb):
        rows = idx[:, 0]
        cols = idx[:, 1]
        cnt = jax.ops.segment_sum(jnp.ones((_E,), jnp.float32), rows,
                                  num_segments=_N)
        acc = jax.ops.segment_sum(jnp.take(X, cols, axis=0), rows,
                                  num_segments=_N)
        return _node0(acc, cnt[:, None], ka, kb)

    def attention(F, u, v, rhat, qs, cs, kas, kbs):
        outs = [F]
        for l in range(2):
            Fsrc = jnp.take(F, src, axis=0).reshape(_EG, _L, _D)
            udst = jnp.take(u[:, 0], dst, axis=0).reshape(_EG, _L)
            vsrc = jnp.take(v[:, 0], src, axis=0).reshape(_EG, _L)
            e, oute = _edge_ac(Fsrc, rhat, udst, vsrc, qs[l], cs[l])
            den = jax.ops.segment_sum(e.reshape(_E), dst, num_segments=_N)
            num = jax.ops.segment_sum(oute.reshape(_E, _D), dst,
                                      num_segments=_N)
            nl = min(l + 1, 1)
            F, u, v = _node_div(num, den[:, None], kas[nl], kbs[nl])
            outs.append(F)
        return jnp.concatenate(outs, axis=1)

    Fer, uer, ver = init_feature(ent_matrix, ent_emb_r, ka_e0, kb_e0)
    Frr, urr, vrr = init_feature(rel_matrix, rel_emb, ka_e0, kb_e0)
    Fet, uet, vet = init_feature(ent_matrix, ent_emb_t, ka_t0, kb_t0)
    Ftt, utt, vtt = init_feature(time_matrix, time_emb, ka_t0, kb_t0)

    ent_r = attention(Fer, uer, ver, rhat_e, (qe0, qe1), (ce0, ce1),
                      (ka_e0, ka_e1), (kb_e0, kb_e1))
    r_f = attention(Frr, urr, vrr, rhat_e, (qe0, qe1), (ce0, ce1),
                    (ka_e0, ka_e1), (kb_e0, kb_e1))
    ent_t = attention(Fet, uet, vet, rhat_t, (qt0, qt1), (ct0, ct1),
                      (ka_t0, ka_t1), (kb_t0, kb_t1))
    t_f = attention(Ftt, utt, vtt, rhat_t, (qt0, qt1), (ct0, ct1),
                    (ka_t0, ka_t1), (kb_t0, kb_t1))

    output_e_r = jnp.concatenate([ent_r, r_f], axis=-1)
    output_e_t = jnp.concatenate([ent_t, t_f], axis=-1)
    return (output_e_r, output_e_t)
